# Initial kernel scaffold; baseline (speedup 1.0000x reference)
import jax
import jax.numpy as jnp
from jax.experimental import pallas as pl


def kernel(mu, x, edge_index, edge_w, W1, W2, W3, W4, W5, W7):
    raise NotImplementedError("write your pallas kernel here")


# speedup vs baseline: 73.6174x; 73.6174x over previous
import functools

import jax
import jax.numpy as jnp
from jax import lax
from jax.experimental import pallas as pl
from jax.experimental.pallas import tpu as pltpu
from jax.experimental.pallas import tpu_sc as plsc

N_NODES = 10000
N_EDGES = 320000
HID = 128
T = 4

NPAD = 10240
ACC = 3 * NPAD
NC, NS, L = 2, 16, 16
EPW = N_EDGES // (NC * NS)
RSL = ACC // NS


def _sc_segment_sums(dst, w):
    mesh = plsc.VectorSubcoreMesh(core_axis_name="c", subcore_axis_name="s")

    @functools.partial(
        pl.kernel,
        mesh=mesh,
        out_type=jax.ShapeDtypeStruct((NC, ACC), jnp.float32),
        compiler_params=pltpu.CompilerParams(needs_layout_passes=False),
        scratch_types=[
            pltpu.VMEM((EPW,), jnp.int32),
            pltpu.VMEM((EPW,), jnp.float32),
            pltpu.VMEM((ACC,), jnp.float32),
            pltpu.VMEM((RSL,), jnp.float32),
            pltpu.VMEM_SHARED((NS, ACC), jnp.float32),
        ],
    )
    def body(dst_hbm, w_hbm, out_hbm, dst_v, w_v, acc_v, tmp_v, partials):
        cid = lax.axis_index("c")
        sid = lax.axis_index("s")
        gid = cid * NS + sid

        def zero(i, _):
            acc_v[pl.ds(i * L, L)] = jnp.zeros((L,), jnp.float32)
            return 0

        lax.fori_loop(0, ACC // L, zero, 0)

        base = gid * EPW
        pltpu.sync_copy(dst_hbm.at[pl.ds(base, EPW)], dst_v)
        pltpu.sync_copy(w_hbm.at[pl.ds(base, EPW)], w_v)

        ones = jnp.ones((L,), jnp.float32)

        def scatter(j, _):
            idx = dst_v[pl.ds(j * L, L)]
            wv = w_v[pl.ds(j * L, L)]
            plsc.addupdate_scatter(acc_v, [idx], ones)
            plsc.addupdate_scatter(acc_v, [idx + NPAD], jnp.maximum(wv, 0.0))
            plsc.addupdate_scatter(acc_v, [idx + 2 * NPAD], jnp.maximum(-wv, 0.0))
            return 0

        lax.fori_loop(0, EPW // L, scatter, 0)

        pltpu.sync_copy(acc_v, partials.at[sid])
        plsc.subcore_barrier()

        sbase = sid * RSL
        pltpu.sync_copy(partials.at[0, pl.ds(sbase, RSL)], tmp_v)

        def red(p, _):
            pltpu.sync_copy(partials.at[p, pl.ds(sbase, RSL)],
                            acc_v.at[pl.ds(0, RSL)])

            def addv(j, _):
                tmp_v[pl.ds(j * L, L)] = (
                    tmp_v[pl.ds(j * L, L)] + acc_v[pl.ds(j * L, L)]
                )
                return 0

            lax.fori_loop(0, RSL // L, addv, 0)
            return 0

        lax.fori_loop(1, NS, red, 0)
        pltpu.sync_copy(tmp_v, out_hbm.at[cid, pl.ds(sbase, RSL)])

    return body(dst, w)


RB = 2000


def _tc_dense(mu, x, deg, swp, swn, W1, W2, W3, W4, W7, w5a, w5b):
    def body(mu_ref, x_ref, deg_ref, swp_ref, swn_ref, W1_ref, W2_ref,
             W3_ref, W4_ref, W7_ref, w5b_ref, out_ref, pool_ref):
        m = mu_ref[...]
        xc = x_ref[...]
        degc = deg_ref[...]
        swpc = swp_ref[...]
        swnc = swn_ref[...]
        for i in range(T):
            w4 = W4_ref[i]
            rp = jnp.dot(jnp.maximum(w4, 0.0), W3_ref[i],
                         preferred_element_type=jnp.float32)
            rn = jnp.dot(jnp.maximum(-w4, 0.0), W3_ref[i],
                         preferred_element_type=jnp.float32)
            b = xc * W1_ref[i] + swpc * rp + swnc * rn
            mm = jnp.dot(m, W2_ref[i], preferred_element_type=jnp.float32)
            m = jnp.maximum(degc * mm + b, 0.0)
        psum = jnp.sum(m, axis=0, keepdims=True)

        @pl.when(pl.program_id(0) == 0)
        def _():
            pool_ref[...] = jnp.zeros_like(pool_ref)

        pool_ref[...] += psum
        nv = jnp.maximum(
            jnp.dot(m, W7_ref[...], preferred_element_type=jnp.float32), 0.0)
        out_ref[...] = jnp.dot(nv, w5b_ref[...],
                               preferred_element_type=jnp.float32)

    nb = N_NODES // RB
    col = pl.BlockSpec((RB, 1), lambda i: (i, 0))
    pout, pool = pl.pallas_call(
        body,
        grid=(nb,),
        in_specs=[
            pl.BlockSpec((RB, HID), lambda i: (i, 0)),
            col, col, col, col,
            pl.BlockSpec((T, 1, HID), lambda i: (0, 0, 0)),
            pl.BlockSpec((T, HID, HID), lambda i: (0, 0, 0)),
            pl.BlockSpec((T, HID, HID), lambda i: (0, 0, 0)),
            pl.BlockSpec((T, 1, HID), lambda i: (0, 0, 0)),
            pl.BlockSpec((HID, HID), lambda i: (0, 0)),
            pl.BlockSpec((HID, 1), lambda i: (0, 0)),
        ],
        out_specs=[
            pl.BlockSpec((RB, 1), lambda i: (i, 0)),
            pl.BlockSpec((1, HID), lambda i: (0, 0)),
        ],
        out_shape=[
            jax.ShapeDtypeStruct((N_NODES, 1), jnp.float32),
            jax.ShapeDtypeStruct((1, HID), jnp.float32),
        ],
    )(mu, x, deg, swp, swn, W1, W2, W3, W4, W7, w5b)

    def fix_body(pout_ref, pool_ref, w5a_ref, out_ref):
        s = jnp.sum(jnp.maximum(pool_ref[...], 0.0) * w5a_ref[...])
        out_ref[...] = pout_ref[...] + s

    return pl.pallas_call(
        fix_body,
        out_shape=jax.ShapeDtypeStruct((N_NODES, 1), jnp.float32),
    )(pout, pool, w5a)


def kernel(mu, x, edge_index, edge_w, W1, W2, W3, W4, W5, W7):
    dst = edge_index[1].astype(jnp.int32)
    w = edge_w[:, 0].astype(jnp.float32)

    partials = _sc_segment_sums(dst, w)
    tot = partials[0] + partials[1]
    deg = tot[:N_NODES][:, None]
    swp = tot[NPAD:NPAD + N_NODES][:, None]
    swn = tot[2 * NPAD:2 * NPAD + N_NODES][:, None]

    w5a = W5[:HID, 0][None, :]
    w5b = W5[HID:]

    out = _tc_dense(mu, x, deg, swp, swn, W1, W2, W3, W4, W7, w5a, w5b)
    return out[:, 0]
